# Initial kernel scaffold; baseline (speedup 1.0000x reference)
import jax
import jax.numpy as jnp
from jax import lax
from jax.experimental import pallas as pl
from jax.experimental.pallas import tpu as pltpu
from jax.experimental.pallas import tpu_sc as plsc

TOKENS = 16384
EXPERTS = 64
K = 8
LANES = 16

NUM_CORES = 2
NUM_SUBCORES = 16
NUM_WORKERS = NUM_CORES * NUM_SUBCORES
TPT = TOKENS // NUM_WORKERS
PAIRS = TPT // 2


def _top8_sorted(x_ref, tok, iota, lane_lt8):
    sorted_groups = []
    for j in range(EXPERTS // LANES):
        keys = x_ref[tok, pl.ds(j * LANES, LANES)]
        vals = iota + j * LANES
        sorted_groups.append(plsc.sort_key_val(keys, vals, descending=True))

    def merge(a, b):
        ak, av = a
        bk, bv = b
        ck = jnp.where(lane_lt8, ak, lax.rev(bk, (0,)))
        cv = jnp.where(lane_lt8, av, lax.rev(bv, (0,)))
        return plsc.sort_key_val(ck, cv, descending=True)

    m01 = merge(sorted_groups[0], sorted_groups[1])
    m23 = merge(sorted_groups[2], sorted_groups[3])
    return merge(m01, m23)


def _body(x_hbm, w_hbm, i_hbm, x_vmem, w_vmem, i_vmem):
    wid = lax.axis_index("s") * NUM_CORES + lax.axis_index("c")
    base = wid * TPT
    pltpu.sync_copy(x_hbm.at[pl.ds(base, TPT)], x_vmem)

    iota = lax.iota(jnp.int32, LANES)
    lane_lt8 = iota < K
    zeros_i = iota * 0
    shift8 = (iota + K) & (LANES - 1)

    def softmax8(fk):
        bmax = fk.at[zeros_i].get(mode="promise_in_bounds")
        z = jnp.where(lane_lt8, jnp.exp(fk - bmax), 0.0)
        s = z
        for sh in (1, 2, 4, 8):
            s = s + s.at[iota ^ sh].get(mode="promise_in_bounds")
        return z / s

    def pair_body(p, carry):
        fk0, fv0 = _top8_sorted(x_vmem, 2 * p, iota, lane_lt8)
        fk1, fv1 = _top8_sorted(x_vmem, 2 * p + 1, iota, lane_lt8)
        w0 = softmax8(fk0)
        w1 = softmax8(fk1)
        w_vmem[p, :] = jnp.where(
            lane_lt8, w0, w1.at[shift8].get(mode="promise_in_bounds"))
        i_vmem[p, :] = jnp.where(
            lane_lt8, fv0, fv1.at[shift8].get(mode="promise_in_bounds"))
        return carry

    lax.fori_loop(0, PAIRS, pair_body, 0)

    obase = wid * PAIRS
    pltpu.sync_copy(w_vmem, w_hbm.at[pl.ds(obase, PAIRS)])
    pltpu.sync_copy(i_vmem, i_hbm.at[pl.ds(obase, PAIRS)])


@jax.jit
def kernel(router_logits):
    mesh = plsc.VectorSubcoreMesh(core_axis_name="c", subcore_axis_name="s")
    w2, i2 = pl.kernel(
        _body,
        out_type=[
            jax.ShapeDtypeStruct((TOKENS // 2, LANES), jnp.float32),
            jax.ShapeDtypeStruct((TOKENS // 2, LANES), jnp.int32),
        ],
        mesh=mesh,
        scratch_types=[
            pltpu.VMEM((TPT, EXPERTS), jnp.float32),
            pltpu.VMEM((PAIRS, LANES), jnp.float32),
            pltpu.VMEM((PAIRS, LANES), jnp.int32),
        ],
    )(router_logits)
    return w2.reshape(TOKENS, K), i2.reshape(TOKENS, K)


# speedup vs baseline: 1.1303x; 1.1303x over previous
import jax
import jax.numpy as jnp
from jax import lax
from jax.experimental import pallas as pl
from jax.experimental.pallas import tpu as pltpu
from jax.experimental.pallas import tpu_sc as plsc

TOKENS = 16384
EXPERTS = 64
K = 8
LANES = 16

NUM_CORES = 2
NUM_SUBCORES = 16
NUM_WORKERS = NUM_CORES * NUM_SUBCORES
TPT = TOKENS // NUM_WORKERS
PAIRS = TPT // 2


def _top8_sorted(x_ref, tok, iota, lane_lt8):
    sorted_groups = []
    for j in range(EXPERTS // LANES):
        keys = x_ref[tok, pl.ds(j * LANES, LANES)]
        vals = iota + j * LANES
        sorted_groups.append(plsc.sort_key_val(keys, vals, descending=True))

    def merge(a, b):
        ak, av = a
        bk, bv = b
        ck = jnp.where(lane_lt8, ak, lax.rev(bk, (0,)))
        cv = jnp.where(lane_lt8, av, lax.rev(bv, (0,)))
        return plsc.sort_key_val(ck, cv, descending=True)

    m01 = merge(sorted_groups[0], sorted_groups[1])
    m23 = merge(sorted_groups[2], sorted_groups[3])
    return merge(m01, m23)


def _body(x_hbm, w_hbm, i_hbm, x_vmem, w_vmem, i_vmem):
    wid = lax.axis_index("s") * NUM_CORES + lax.axis_index("c")
    base = wid * TPT
    pltpu.sync_copy(x_hbm.at[pl.ds(base, TPT)], x_vmem)

    iota = lax.iota(jnp.int32, LANES)
    lane_lt8 = iota < K
    zeros_i = iota * 0
    shift8 = (iota + K) & (LANES - 1)

    def softmax8(fk):
        bmax = fk.at[zeros_i].get(mode="promise_in_bounds")
        z = jnp.where(lane_lt8, jnp.exp(fk - bmax), 0.0)
        s = z
        for sh in (1, 2, 4, 8):
            s = s + s.at[iota ^ sh].get(mode="promise_in_bounds")
        return z / s

    def pair_body(p, carry):
        fk0, fv0 = _top8_sorted(x_vmem, 2 * p, iota, lane_lt8)
        fk1, fv1 = _top8_sorted(x_vmem, 2 * p + 1, iota, lane_lt8)
        w0 = softmax8(fk0)
        w1 = softmax8(fk1)
        w_vmem[p, :] = jnp.where(
            lane_lt8, w0, w1.at[shift8].get(mode="promise_in_bounds"))
        i_vmem[p, :] = jnp.where(
            lane_lt8, fv0, fv1.at[shift8].get(mode="promise_in_bounds"))
        return carry

    lax.fori_loop(0, PAIRS, pair_body, 0)

    obase = wid * PAIRS
    pltpu.sync_copy(w_vmem, w_hbm.at[pl.ds(obase, PAIRS)])
    pltpu.sync_copy(i_vmem, i_hbm.at[pl.ds(obase, PAIRS)])


@jax.jit
def kernel(router_logits):
    mesh = plsc.VectorSubcoreMesh(core_axis_name="c", subcore_axis_name="s")
    w2, i2 = pl.kernel(
        _body,
        out_type=[
            jax.ShapeDtypeStruct((TOKENS // 2, LANES), jnp.float32),
            jax.ShapeDtypeStruct((TOKENS // 2, LANES), jnp.int32),
        ],
        mesh=mesh,
        compiler_params=pltpu.CompilerParams(needs_layout_passes=False),
        scratch_types=[
            pltpu.VMEM((TPT, EXPERTS), jnp.float32),
            pltpu.VMEM((PAIRS, LANES), jnp.float32),
            pltpu.VMEM((PAIRS, LANES), jnp.int32),
        ],
    )(router_logits)
    return w2.reshape(TOKENS, K), i2.reshape(TOKENS, K)


# speedup vs baseline: 1.3496x; 1.1940x over previous
import jax
import jax.numpy as jnp
from jax import lax
from jax.experimental import pallas as pl
from jax.experimental.pallas import tpu as pltpu
from jax.experimental.pallas import tpu_sc as plsc

TOKENS = 16384
EXPERTS = 64
K = 8
LANES = 16

NUM_CORES = 2
NUM_SUBCORES = 16
NUM_WORKERS = NUM_CORES * NUM_SUBCORES
TPT = TOKENS // NUM_WORKERS
PAIRS = TPT // 2


def _top8_sorted(x_ref, tok, iota, lane_lt8):
    sorted_groups = []
    for j in range(EXPERTS // LANES):
        keys = x_ref[tok, pl.ds(j * LANES, LANES)]
        vals = iota + j * LANES
        sorted_groups.append(plsc.sort_key_val(keys, vals, descending=True))

    def merge(a, b):
        ak, av = a
        bk, bv = b
        ck = jnp.where(lane_lt8, ak, lax.rev(bk, (0,)))
        cv = jnp.where(lane_lt8, av, lax.rev(bv, (0,)))
        return plsc.sort_key_val(ck, cv, descending=True)

    m01 = merge(sorted_groups[0], sorted_groups[1])
    m23 = merge(sorted_groups[2], sorted_groups[3])
    return merge(m01, m23)


def _body(x_hbm, w_hbm, i_hbm, x_vmem, w_vmem, i_vmem):
    wid = lax.axis_index("s") * NUM_CORES + lax.axis_index("c")
    base = wid * TPT
    pltpu.sync_copy(x_hbm.at[pl.ds(base, TPT)], x_vmem)

    iota = lax.iota(jnp.int32, LANES)
    lane_lt8 = iota < K
    zeros_i = iota * 0
    shift8 = (iota + K) & (LANES - 1)

    def softmax8(fk):
        bmax = fk.at[zeros_i].get(mode="promise_in_bounds")
        z = jnp.where(lane_lt8, jnp.exp(fk - bmax), 0.0)
        s = z
        for sh in (1, 2, 4, 8):
            s = s + s.at[iota ^ sh].get(mode="promise_in_bounds")
        return z / s

    @plsc.parallel_loop(0, PAIRS, step=1, unroll=4)
    def pair_body(p):
        fk0, fv0 = _top8_sorted(x_vmem, 2 * p, iota, lane_lt8)
        fk1, fv1 = _top8_sorted(x_vmem, 2 * p + 1, iota, lane_lt8)
        w0 = softmax8(fk0)
        w1 = softmax8(fk1)
        w_vmem[p, :] = jnp.where(
            lane_lt8, w0, w1.at[shift8].get(mode="promise_in_bounds"))
        i_vmem[p, :] = jnp.where(
            lane_lt8, fv0, fv1.at[shift8].get(mode="promise_in_bounds"))

    obase = wid * PAIRS
    pltpu.sync_copy(w_vmem, w_hbm.at[pl.ds(obase, PAIRS)])
    pltpu.sync_copy(i_vmem, i_hbm.at[pl.ds(obase, PAIRS)])


@jax.jit
def kernel(router_logits):
    mesh = plsc.VectorSubcoreMesh(core_axis_name="c", subcore_axis_name="s")
    w2, i2 = pl.kernel(
        _body,
        out_type=[
            jax.ShapeDtypeStruct((TOKENS // 2, LANES), jnp.float32),
            jax.ShapeDtypeStruct((TOKENS // 2, LANES), jnp.int32),
        ],
        mesh=mesh,
        compiler_params=pltpu.CompilerParams(needs_layout_passes=False),
        scratch_types=[
            pltpu.VMEM((TPT, EXPERTS), jnp.float32),
            pltpu.VMEM((PAIRS, LANES), jnp.float32),
            pltpu.VMEM((PAIRS, LANES), jnp.int32),
        ],
    )(router_logits)
    return w2.reshape(TOKENS, K), i2.reshape(TOKENS, K)


# speedup vs baseline: 1.3549x; 1.0039x over previous
import jax
import jax.numpy as jnp
from jax import lax
from jax.experimental import pallas as pl
from jax.experimental.pallas import tpu as pltpu
from jax.experimental.pallas import tpu_sc as plsc

TOKENS = 16384
EXPERTS = 64
K = 8
LANES = 16

NUM_CORES = 2
NUM_SUBCORES = 16
NUM_WORKERS = NUM_CORES * NUM_SUBCORES
TPT = TOKENS // NUM_WORKERS
PAIRS = TPT // 2


def _top8_sorted(x_ref, tok, iota, lane_lt8):
    shift8 = (iota + K) & (LANES - 1)
    sorted_groups = []
    for j in range(EXPERTS // LANES):
        keys = x_ref[tok, pl.ds(j * LANES, LANES)]
        vals = iota + j * LANES
        sorted_groups.append(plsc.sort_key_val(keys, vals, descending=True))

    def merge(a, b):
        ak, av = a
        bk, bv = b
        ck = jnp.where(lane_lt8, ak, bk.at[shift8].get(mode="promise_in_bounds"))
        cv = jnp.where(lane_lt8, av, bv.at[shift8].get(mode="promise_in_bounds"))
        return plsc.sort_key_val(ck, cv, descending=True)

    m01 = merge(sorted_groups[0], sorted_groups[1])
    m23 = merge(sorted_groups[2], sorted_groups[3])
    return merge(m01, m23)


def _body(x_hbm, w_hbm, i_hbm, x_vmem, w_vmem, i_vmem):
    wid = lax.axis_index("s") * NUM_CORES + lax.axis_index("c")
    base = wid * TPT
    pltpu.sync_copy(x_hbm.at[pl.ds(base, TPT)], x_vmem)

    iota = lax.iota(jnp.int32, LANES)
    lane_lt8 = iota < K
    zeros_i = iota * 0
    shift8 = (iota + K) & (LANES - 1)

    def softmax8(fk):
        bmax = fk.at[zeros_i].get(mode="promise_in_bounds")
        z = jnp.where(lane_lt8, jnp.exp(fk - bmax), 0.0)
        s = z
        for sh in (1, 2, 4, 8):
            s = s + s.at[iota ^ sh].get(mode="promise_in_bounds")
        return z / s

    @plsc.parallel_loop(0, PAIRS, step=1, unroll=4)
    def pair_body(p):
        fk0, fv0 = _top8_sorted(x_vmem, 2 * p, iota, lane_lt8)
        fk1, fv1 = _top8_sorted(x_vmem, 2 * p + 1, iota, lane_lt8)
        w0 = softmax8(fk0)
        w1 = softmax8(fk1)
        w_vmem[p, :] = jnp.where(
            lane_lt8, w0, w1.at[shift8].get(mode="promise_in_bounds"))
        i_vmem[p, :] = jnp.where(
            lane_lt8, fv0, fv1.at[shift8].get(mode="promise_in_bounds"))

    obase = wid * PAIRS
    pltpu.sync_copy(w_vmem, w_hbm.at[pl.ds(obase, PAIRS)])
    pltpu.sync_copy(i_vmem, i_hbm.at[pl.ds(obase, PAIRS)])


@jax.jit
def kernel(router_logits):
    mesh = plsc.VectorSubcoreMesh(core_axis_name="c", subcore_axis_name="s")
    w2, i2 = pl.kernel(
        _body,
        out_type=[
            jax.ShapeDtypeStruct((TOKENS // 2, LANES), jnp.float32),
            jax.ShapeDtypeStruct((TOKENS // 2, LANES), jnp.int32),
        ],
        mesh=mesh,
        compiler_params=pltpu.CompilerParams(needs_layout_passes=False),
        scratch_types=[
            pltpu.VMEM((TPT, EXPERTS), jnp.float32),
            pltpu.VMEM((PAIRS, LANES), jnp.float32),
            pltpu.VMEM((PAIRS, LANES), jnp.int32),
        ],
    )(router_logits)
    return w2.reshape(TOKENS, K), i2.reshape(TOKENS, K)


# speedup vs baseline: 1.5606x; 1.1519x over previous
import jax
import jax.numpy as jnp
from jax import lax
from jax.experimental import pallas as pl
from jax.experimental.pallas import tpu as pltpu
from jax.experimental.pallas import tpu_sc as plsc

TOKENS = 16384
EXPERTS = 64
K = 8
LANES = 16

NUM_CORES = 2
NUM_SUBCORES = 16
NUM_WORKERS = NUM_CORES * NUM_SUBCORES
TPT = TOKENS // NUM_WORKERS
PAIRS = TPT // 2


def _top8_sorted(x_ref, tok, iota, lane_lt8):
    shift8 = (iota + K) & (LANES - 1)
    sorted_groups = []
    for j in range(EXPERTS // LANES):
        keys = x_ref[tok, pl.ds(j * LANES, LANES)]
        vals = iota + j * LANES
        sorted_groups.append(plsc.sort_key_val(keys, vals, descending=True))

    def merge(a, b):
        ak, av = a
        bk, bv = b
        ck = jnp.where(lane_lt8, ak, bk.at[shift8].get(mode="promise_in_bounds"))
        cv = jnp.where(lane_lt8, av, bv.at[shift8].get(mode="promise_in_bounds"))
        return plsc.sort_key_val(ck, cv, descending=True)

    m01 = merge(sorted_groups[0], sorted_groups[1])
    m23 = merge(sorted_groups[2], sorted_groups[3])
    return merge(m01, m23)


CHUNK = 256
CHUNK_PAIRS = CHUNK // 2


def _body(x_hbm, w_hbm, i_hbm, x_vmem, w_vmem, i_vmem):
    wid = lax.axis_index("s") * NUM_CORES + lax.axis_index("c")
    base = wid * TPT

    iota = lax.iota(jnp.int32, LANES)
    lane_lt8 = iota < K
    zeros_i = iota * 0
    shift8 = (iota + K) & (LANES - 1)

    def softmax8(fk):
        bmax = fk.at[zeros_i].get(mode="promise_in_bounds")
        z = jnp.where(lane_lt8, jnp.exp(fk - bmax), 0.0)
        s = z
        for sh in (1, 2, 4, 8):
            s = s + s.at[iota ^ sh].get(mode="promise_in_bounds")
        return z / s

    col8 = iota & (K - 1)
    lane_ge8 = (iota >> 3) & 1

    for chunk in range(TPT // CHUNK):
        cbase = base + chunk * CHUNK
        pltpu.sync_copy(x_hbm.at[pl.ds(cbase, CHUNK)], x_vmem)

        @plsc.parallel_loop(0, CHUNK_PAIRS, step=1, unroll=4)
        def pair_body(p):
            fk0, fv0 = _top8_sorted(x_vmem, 2 * p, iota, lane_lt8)
            fk1, fv1 = _top8_sorted(x_vmem, 2 * p + 1, iota, lane_lt8)
            w0 = softmax8(fk0)
            w1 = softmax8(fk1)
            wp = jnp.where(
                lane_lt8, w0, w1.at[shift8].get(mode="promise_in_bounds"))
            ip = jnp.where(
                lane_lt8, fv0, fv1.at[shift8].get(mode="promise_in_bounds"))
            rows = 2 * p + lane_ge8
            plsc.store_scatter(w_vmem, [rows, col8], wp)
            plsc.store_scatter(i_vmem, [rows, col8], ip)

        pltpu.sync_copy(w_vmem, w_hbm.at[pl.ds(cbase, CHUNK)])
        pltpu.sync_copy(i_vmem, i_hbm.at[pl.ds(cbase, CHUNK)])


@jax.jit
def kernel(router_logits):
    mesh = plsc.VectorSubcoreMesh(core_axis_name="c", subcore_axis_name="s")
    w2, i2 = pl.kernel(
        _body,
        out_type=[
            jax.ShapeDtypeStruct((TOKENS, K), jnp.float32),
            jax.ShapeDtypeStruct((TOKENS, K), jnp.int32),
        ],
        mesh=mesh,
        compiler_params=pltpu.CompilerParams(needs_layout_passes=False),
        scratch_types=[
            pltpu.VMEM((CHUNK, EXPERTS), jnp.float32),
            pltpu.VMEM((CHUNK, K), jnp.float32),
            pltpu.VMEM((CHUNK, K), jnp.int32),
        ],
    )(router_logits)
    return w2, i2


# speedup vs baseline: 1.9253x; 1.2336x over previous
import jax
import jax.numpy as jnp
from jax import lax
from jax.experimental import pallas as pl
from jax.experimental.pallas import tpu as pltpu
from jax.experimental.pallas import tpu_sc as plsc

TOKENS = 16384
EXPERTS = 64
K = 8
LANES = 16

NUM_CORES = 2
NUM_SUBCORES = 16
NUM_WORKERS = NUM_CORES * NUM_SUBCORES
TPT = TOKENS // NUM_WORKERS


def _body(xt_hbm, w_hbm, i_hbm, x_vmem, w_vmem, i_vmem):
    wid = lax.axis_index("s") * NUM_CORES + lax.axis_index("c")
    base = wid * TPT
    pltpu.sync_copy(xt_hbm.at[:, pl.ds(base, TPT)], x_vmem)

    iota = lax.iota(jnp.int32, LANES)
    lane_lt8 = iota < K
    zeros_i = iota * 0
    shift8 = (iota + K) & (LANES - 1)

    def merge(a, b):
        ak, av = a
        bk, bv = b
        ck = jnp.where(lane_lt8, ak, bk.at[shift8].get(mode="promise_in_bounds"))
        cv = jnp.where(lane_lt8, av, bv.at[shift8].get(mode="promise_in_bounds"))
        return plsc.sort_key_val(ck, cv, descending=True)

    @plsc.parallel_loop(0, TPT, step=1, unroll=4)
    def tok_body(t):
        tcol = jnp.full((LANES,), t, jnp.int32)
        groups = []
        for j in range(EXPERTS // LANES):
            rows = iota + j * LANES
            keys = plsc.load_gather(x_vmem, [rows, tcol])
            groups.append(plsc.sort_key_val(keys, rows, descending=True))
        m01 = merge(groups[0], groups[1])
        m23 = merge(groups[2], groups[3])
        fk, fv = merge(m01, m23)

        bmax = fk.at[zeros_i].get(mode="promise_in_bounds")
        z = jnp.where(lane_lt8, jnp.exp(fk - bmax), 0.0)
        s = z
        for sh in (1, 2, 4):
            s = s + s.at[iota ^ sh].get(mode="promise_in_bounds")
        w = z / s

        plsc.store_scatter(w_vmem, [iota, tcol], w, mask=lane_lt8)
        plsc.store_scatter(i_vmem, [iota, tcol], fv, mask=lane_lt8)

    pltpu.sync_copy(w_vmem, w_hbm.at[:, pl.ds(base, TPT)])
    pltpu.sync_copy(i_vmem, i_hbm.at[:, pl.ds(base, TPT)])


@jax.jit
def kernel(router_logits):
    mesh = plsc.VectorSubcoreMesh(core_axis_name="c", subcore_axis_name="s")
    wt, it = pl.kernel(
        _body,
        out_type=[
            jax.ShapeDtypeStruct((K, TOKENS), jnp.float32),
            jax.ShapeDtypeStruct((K, TOKENS), jnp.int32),
        ],
        mesh=mesh,
        compiler_params=pltpu.CompilerParams(needs_layout_passes=False),
        scratch_types=[
            pltpu.VMEM((EXPERTS, TPT), jnp.float32),
            pltpu.VMEM((K, TPT), jnp.float32),
            pltpu.VMEM((K, TPT), jnp.int32),
        ],
    )(router_logits.T)
    return wt.T, it.T
